# baseline (speedup 1.0000x reference)
import functools

import jax
import jax.numpy as jnp
import numpy as np
from jax import lax
from jax.experimental import pallas as pl
from jax.experimental.pallas import tpu as pltpu
from jax.experimental.pallas import tpu_sc as plsc

_BATCH = 16
_PATCHES = 2048
_EMBED = 1024
_MASK_RATIO = 0.75
_NUM_MASKED = int(_MASK_RATIO * _PATCHES)
_ROWS = _BATCH * _PATCHES

_NW = 32
_CHUNK = 32
_M_PER_W = _BATCH * _NUM_MASKED // _NW
_K_PER_W = (_ROWS - _BATCH * _NUM_MASKED) // _NW
_M_CHUNKS = _M_PER_W // _CHUNK
_K_CHUNKS = _K_PER_W // _CHUNK


@functools.lru_cache(maxsize=1)
def _static_mask():
    cpu = jax.local_devices(backend="cpu")[0]
    with jax.ensure_compile_time_eval(), jax.default_device(cpu):
        pkey = jax.random.key(42)
        keys = jax.random.split(pkey, _BATCH)
        perms = jax.vmap(lambda k: jax.random.permutation(k, _PATCHES))(keys)
        perms = np.asarray(perms)
    mask_indices = perms[:, :_NUM_MASKED].astype(np.int32)
    base = (np.arange(_BATCH, dtype=np.int32) * _PATCHES)[:, None]
    masked_gid = (base + mask_indices).reshape(-1)
    kept_gid = (base + perms[:, _NUM_MASKED:].astype(np.int32)).reshape(-1)
    midx = masked_gid.reshape(_NW, _M_CHUNKS, _CHUNK)
    kidx = kept_gid.reshape(_NW, _K_CHUNKS, _CHUNK)
    return mask_indices, midx, kidx


def _sc_body(x_hbm, midx_hbm, kidx_hbm, zsrc_hbm, out_hbm,
             midx_v, kidx_v, zbuf, rbuf_a, rbuf_b, sem_z, sem_g, sem_s):
    wid = lax.axis_index("s") * 2 + lax.axis_index("c")
    pltpu.sync_copy(midx_hbm.at[wid], midx_v)
    pltpu.sync_copy(kidx_hbm.at[wid], kidx_v)
    pltpu.sync_copy(zsrc_hbm, zbuf)
    zcopies = [
        pltpu.make_async_copy(zbuf, out_hbm.at[midx_v.at[c]], sem_z)
        for c in range(_M_CHUNKS)
    ]
    for cp in zcopies:
        cp.start()
    rbufs = [rbuf_a, rbuf_b]
    gathers = [
        pltpu.make_async_copy(x_hbm.at[kidx_v.at[c]], rbufs[c % 2], sem_g)
        for c in range(_K_CHUNKS)
    ]
    scatters = [
        pltpu.make_async_copy(rbufs[c % 2], out_hbm.at[kidx_v.at[c]], sem_s)
        for c in range(_K_CHUNKS)
    ]
    gathers[0].start()
    for c in range(_K_CHUNKS):
        gathers[c].wait()
        scatters[c].start()
        if c + 1 < _K_CHUNKS:
            if c >= 1:
                scatters[c - 1].wait()
            gathers[c + 1].start()
    scatters[_K_CHUNKS - 1].wait()
    for cp in zcopies:
        cp.wait()


def kernel(x):
    mask_indices, midx, kidx = _static_mask()
    x2 = x.reshape(_ROWS, _EMBED)
    mesh = plsc.VectorSubcoreMesh(core_axis_name="c", subcore_axis_name="s")
    sc_call = functools.partial(
        pl.kernel,
        mesh=mesh,
        out_type=jax.ShapeDtypeStruct((_ROWS, _EMBED), jnp.float32),
        scratch_types=[
            pltpu.VMEM((_M_CHUNKS, _CHUNK), jnp.int32),
            pltpu.VMEM((_K_CHUNKS, _CHUNK), jnp.int32),
            pltpu.VMEM((_CHUNK, _EMBED), jnp.float32),
            pltpu.VMEM((_CHUNK, _EMBED), jnp.float32),
            pltpu.VMEM((_CHUNK, _EMBED), jnp.float32),
            pltpu.SemaphoreType.DMA,
            pltpu.SemaphoreType.DMA,
            pltpu.SemaphoreType.DMA,
        ],
    )(_sc_body)
    masked2 = sc_call(
        x2,
        jnp.asarray(midx),
        jnp.asarray(kidx),
        jnp.zeros((_CHUNK, _EMBED), jnp.float32),
    )
    masked_x = masked2.reshape(_BATCH, _PATCHES, _EMBED)
    return (masked_x, jnp.asarray(mask_indices), x)


# speedup vs baseline: 1.0288x; 1.0288x over previous
import functools

import jax
import jax.numpy as jnp
import numpy as np
from jax import lax
from jax.experimental import pallas as pl
from jax.experimental.pallas import tpu as pltpu
from jax.experimental.pallas import tpu_sc as plsc

_BATCH = 16
_PATCHES = 2048
_EMBED = 1024
_MASK_RATIO = 0.75
_NUM_MASKED = int(_MASK_RATIO * _PATCHES)
_ROWS = _BATCH * _PATCHES

_NW = 32
_CHUNK = 32
_M_PER_W = _BATCH * _NUM_MASKED // _NW
_K_PER_W = (_ROWS - _BATCH * _NUM_MASKED) // _NW
_M_CHUNKS = _M_PER_W // _CHUNK
_K_CHUNKS = _K_PER_W // _CHUNK


@functools.lru_cache(maxsize=1)
def _static_mask():
    cpu = jax.local_devices(backend="cpu")[0]
    with jax.ensure_compile_time_eval(), jax.default_device(cpu):
        pkey = jax.random.key(42)
        keys = jax.random.split(pkey, _BATCH)
        perms = jax.vmap(lambda k: jax.random.permutation(k, _PATCHES))(keys)
        perms = np.asarray(perms)
    mask_indices = perms[:, :_NUM_MASKED].astype(np.int32)
    base = (np.arange(_BATCH, dtype=np.int32) * _PATCHES)[:, None]
    masked_gid = (base + mask_indices).reshape(-1)
    kept_gid = (base + perms[:, _NUM_MASKED:].astype(np.int32)).reshape(-1)
    midx = masked_gid.reshape(_NW, _M_CHUNKS, _CHUNK)
    kidx = kept_gid.reshape(_NW, _K_CHUNKS, _CHUNK)
    return mask_indices, midx, kidx


def _sc_body(x_hbm, midx_hbm, kidx_hbm, zsrc_hbm, out_hbm,
             midx_v, kidx_v, zbuf, rbuf_a, rbuf_b, sem_z, sem_g, sem_s):
    wid = lax.axis_index("s") * 2 + lax.axis_index("c")
    pltpu.sync_copy(midx_hbm.at[wid], midx_v)
    pltpu.sync_copy(kidx_hbm.at[wid], kidx_v)
    pltpu.sync_copy(zsrc_hbm, zbuf)
    zcopies = [
        pltpu.make_async_copy(zbuf, out_hbm.at[midx_v.at[c]], sem_z)
        for c in range(_M_CHUNKS)
    ]
    for cp in zcopies:
        cp.start()
    rbufs = [rbuf_a, rbuf_b]
    gathers = [
        pltpu.make_async_copy(x_hbm.at[kidx_v.at[c]], rbufs[c % 2], sem_g)
        for c in range(_K_CHUNKS)
    ]
    scatters = [
        pltpu.make_async_copy(rbufs[c % 2], out_hbm.at[kidx_v.at[c]], sem_s)
        for c in range(_K_CHUNKS)
    ]
    gathers[0].start()
    for c in range(_K_CHUNKS):
        gathers[c].wait()
        scatters[c].start()
        if c + 1 < _K_CHUNKS:
            if c >= 1:
                scatters[c - 1].wait()
            gathers[c + 1].start()
    scatters[_K_CHUNKS - 1].wait()
    for cp in zcopies:
        cp.wait()


def _copy_kernel(x_ref, c_ref):
    c_ref[0] = x_ref[0]


def kernel(x):
    mask_indices, midx, kidx = _static_mask()
    x2 = x.reshape(_ROWS, _EMBED)
    mesh = plsc.VectorSubcoreMesh(core_axis_name="c", subcore_axis_name="s")
    sc_call = functools.partial(
        pl.kernel,
        mesh=mesh,
        out_type=jax.ShapeDtypeStruct((_ROWS, _EMBED), jnp.float32),
        scratch_types=[
            pltpu.VMEM((_M_CHUNKS, _CHUNK), jnp.int32),
            pltpu.VMEM((_K_CHUNKS, _CHUNK), jnp.int32),
            pltpu.VMEM((_CHUNK, _EMBED), jnp.float32),
            pltpu.VMEM((_CHUNK, _EMBED), jnp.float32),
            pltpu.VMEM((_CHUNK, _EMBED), jnp.float32),
            pltpu.SemaphoreType.DMA,
            pltpu.SemaphoreType.DMA,
            pltpu.SemaphoreType.DMA,
        ],
    )(_sc_body)
    masked2 = sc_call(
        x2,
        jnp.asarray(midx),
        jnp.asarray(kidx),
        jnp.zeros((_CHUNK, _EMBED), jnp.float32),
    )
    masked_x = masked2.reshape(_BATCH, _PATCHES, _EMBED)
    x_original = pl.pallas_call(
        _copy_kernel,
        grid=(_BATCH,),
        in_specs=[pl.BlockSpec((1, _PATCHES, _EMBED), lambda i: (i, 0, 0))],
        out_specs=pl.BlockSpec((1, _PATCHES, _EMBED), lambda i: (i, 0, 0)),
        out_shape=jax.ShapeDtypeStruct((_BATCH, _PATCHES, _EMBED), jnp.float32),
    )(x)
    return (masked_x, jnp.asarray(mask_indices), x_original)


# speedup vs baseline: 1.3377x; 1.3003x over previous
import functools

import jax
import jax.numpy as jnp
import numpy as np
from jax.experimental import pallas as pl
from jax.experimental.pallas import tpu as pltpu

_BATCH = 16
_PATCHES = 2048
_EMBED = 1024
_MASK_RATIO = 0.75
_NUM_MASKED = int(_MASK_RATIO * _PATCHES)


@functools.lru_cache(maxsize=1)
def _static_mask():
    cpu = jax.local_devices(backend="cpu")[0]
    with jax.ensure_compile_time_eval(), jax.default_device(cpu):
        pkey = jax.random.key(42)
        keys = jax.random.split(pkey, _BATCH)
        perms = jax.vmap(lambda k: jax.random.permutation(k, _PATCHES))(keys)
        perms = np.asarray(perms)
    mask_indices = perms[:, :_NUM_MASKED].astype(np.int32)
    keep = np.ones((_BATCH, _PATCHES), dtype=np.int8)
    keep[np.arange(_BATCH)[:, None], mask_indices] = 0
    return mask_indices, keep


def _mask_kernel(x_ref, m_ref, o_ref, c_ref, sem):
    i = pl.program_id(0)
    cp = pltpu.make_async_copy(x_ref, c_ref.at[pl.ds(i, 1)], sem)
    cp.start()
    o_ref[0] = x_ref[0] * m_ref[0].astype(jnp.float32)
    cp.wait()


def kernel(x):
    mask_indices, keep = _static_mask()
    keep3 = jnp.asarray(keep.reshape(_BATCH, _PATCHES, 1))
    masked_x, x_original = pl.pallas_call(
        _mask_kernel,
        grid=(_BATCH,),
        in_specs=[
            pl.BlockSpec((1, _PATCHES, _EMBED), lambda i: (i, 0, 0)),
            pl.BlockSpec((1, _PATCHES, 1), lambda i: (i, 0, 0)),
        ],
        out_specs=[
            pl.BlockSpec((1, _PATCHES, _EMBED), lambda i: (i, 0, 0)),
            pl.BlockSpec(memory_space=pl.ANY),
        ],
        out_shape=[
            jax.ShapeDtypeStruct((_BATCH, _PATCHES, _EMBED), jnp.float32),
            jax.ShapeDtypeStruct((_BATCH, _PATCHES, _EMBED), jnp.float32),
        ],
        scratch_shapes=[pltpu.SemaphoreType.DMA],
    )(x, keep3)
    return (masked_x, jnp.asarray(mask_indices), x_original)
